# baseline (speedup 1.0000x reference)
import functools

import jax
import jax.numpy as jnp
from jax import lax
from jax.experimental import pallas as pl
from jax.experimental.pallas import tpu as pltpu
from jax.experimental.pallas import tpu_sc as plsc

_B, _N, _DM = 4, 4096, 1024
_NS = 2
_SD = _DM // _NS
_K = 8192
_BETA = 0.25
_T = _B * _N

_TM = 512
_TN = 2736
_KPAD = 3 * _TN
_TB = _T // _TM
_KB = 3

_NW = 32
_TOK_W = _T // _NW
_CH = 128
_NCH = _TOK_W // _CH


def _argmin_body(xt_ref, w_ref, fs_ref, wsq_ref, ids_ref, mind_ref,
                 rmin_ref, ridx_ref):
    k = pl.program_id(0)
    t = pl.program_id(1)
    mm = lax.dot_general(
        w_ref[...], xt_ref[...],
        dimension_numbers=(((1,), (0,)), ((), ())),
        preferred_element_type=jnp.float32)
    t1 = wsq_ref[:, :] + fs_ref[0]
    d = t1 - 2.0 * mm
    bmin = jnp.min(d, axis=0, keepdims=True)
    gidx = jax.lax.broadcasted_iota(jnp.int32, (_TN, _TM), 0) + k * _TN
    cand = jnp.where(d == bmin, gidx, jnp.int32(_KPAD))
    bidx = jnp.min(cand, axis=0, keepdims=True)

    is_first = k == 0
    is_last = k == pl.num_programs(0) - 1
    sl = pl.ds(t * _TM, _TM)
    prev_min = rmin_ref[:, sl]
    prev_idx = ridx_ref[:, sl]
    take_new = jnp.logical_or(is_first, bmin < prev_min)
    val = jnp.where(take_new, bmin, prev_min)
    idx = jnp.where(take_new, bidx, prev_idx)
    rounded = val.astype(jnp.bfloat16).astype(jnp.float32)
    rmin_ref[:, sl] = jnp.where(is_last, val, rounded)
    ridx_ref[:, sl] = idx

    @pl.when(is_last)
    def _():
        ids_ref[0] = idx
        mind_ref[0] = val


def _argmin_call(xt_s, w_p, fs3, wsq2):
    return pl.pallas_call(
        _argmin_body,
        grid=(_KB, _TB),
        in_specs=[
            pl.BlockSpec((_SD, _TM), lambda k, t: (0, t)),
            pl.BlockSpec((_TN, _SD), lambda k, t: (k, 0)),
            pl.BlockSpec((1, 1, _TM), lambda k, t: (t, 0, 0)),
            pl.BlockSpec((_TN, 1), lambda k, t: (k, 0)),
        ],
        out_specs=[
            pl.BlockSpec((1, 1, _TM), lambda k, t: (t, 0, 0)),
            pl.BlockSpec((1, 1, _TM), lambda k, t: (t, 0, 0)),
        ],
        out_shape=[
            jax.ShapeDtypeStruct((_TB, 1, _TM), jnp.int32),
            jax.ShapeDtypeStruct((_TB, 1, _TM), jnp.float32),
        ],
        compiler_params=pltpu.CompilerParams(
            dimension_semantics=("arbitrary", "arbitrary")),
        scratch_shapes=[
            pltpu.VMEM((1, _T), jnp.float32),
            pltpu.VMEM((1, _T), jnp.int32),
        ],
    )(xt_s, w_p, fs3, wsq2)


def _gather_sc(w, ids):
    mesh = plsc.VectorSubcoreMesh(core_axis_name="c", subcore_axis_name="s")

    @functools.partial(
        pl.kernel,
        mesh=mesh,
        out_type=jax.ShapeDtypeStruct((_T, _SD), jnp.float32),
        scratch_types=[
            pltpu.VMEM((_CH,), jnp.int32),
            pltpu.VMEM((_CH, _SD), jnp.float32),
            pltpu.SemaphoreType.DMA,
        ],
    )
    def gk(w_hbm, i_hbm, z_hbm, idx_v, rows_v, sem):
        wid = lax.axis_index("s") * 2 + lax.axis_index("c")
        base = wid * _TOK_W
        for c in range(_NCH):
            off = base + c * _CH
            pltpu.sync_copy(i_hbm.at[pl.ds(off, _CH)], idx_v)
            pltpu.async_copy(w_hbm.at[idx_v], rows_v, sem).wait()
            pltpu.sync_copy(rows_v, z_hbm.at[pl.ds(off, _CH)])

    return gk(w, ids)


def _st_body(h_ref, z0_ref, z1_ref, z_ref):
    h0 = h_ref[:, :_SD]
    h1 = h_ref[:, _SD:]
    z_ref[:, :_SD] = h0 + (z0_ref[...] - h0)
    z_ref[:, _SD:] = h1 + (z1_ref[...] - h1)


def _st_call(h2, z0, z1):
    tmz = 1024
    return pl.pallas_call(
        _st_body,
        grid=(_T // tmz,),
        in_specs=[
            pl.BlockSpec((tmz, _DM), lambda i: (i, 0)),
            pl.BlockSpec((tmz, _SD), lambda i: (i, 0)),
            pl.BlockSpec((tmz, _SD), lambda i: (i, 0)),
        ],
        out_specs=pl.BlockSpec((tmz, _DM), lambda i: (i, 0)),
        out_shape=jax.ShapeDtypeStruct((_T, _DM), jnp.float32),
    )(h2, z0, z1)


def kernel(h, W0, W1):
    hf = h.reshape(_T, _NS, _SD)
    xt = jnp.transpose(hf, (1, 2, 0))
    w0p = jnp.pad(W0, ((0, _KPAD - _K), (0, 0)))
    w1p = jnp.pad(W1, ((0, _KPAD - _K), (0, 0)))
    f0 = jnp.sum(hf[:, 0, :] ** 2, axis=1)
    f1 = jnp.sum(hf[:, 1, :] ** 2, axis=1)
    wsq0 = jnp.pad(jnp.sum(W0 ** 2, axis=1), (0, _KPAD - _K),
                   constant_values=jnp.inf).reshape(_KPAD, 1)
    wsq1 = jnp.pad(jnp.sum(W1 ** 2, axis=1), (0, _KPAD - _K),
                   constant_values=jnp.inf).reshape(_KPAD, 1)

    ids0_3, mind0 = _argmin_call(xt[0], w0p, f0.reshape(_TB, 1, _TM), wsq0)
    ids0 = ids0_3.reshape(_T)
    z0 = _gather_sc(W0, ids0)
    ids1_3, mind1 = _argmin_call(xt[1], w1p, f1.reshape(_TB, 1, _TM), wsq1)
    ids1 = ids1_3.reshape(_T)
    z1 = _gather_sc(W1, ids1)

    z = _st_call(h.reshape(_T, _DM), z0, z1).reshape(_B, _N, _DM)
    ids_packed = (ids0 + _K * ids1).reshape(_B, _N)
    vq_total = (1.0 + _BETA) * ((jnp.sum(mind0) + jnp.sum(mind1))
                                / (_T * _SD))
    return (z, ids_packed, vq_total)


# speedup vs baseline: 1.0432x; 1.0432x over previous
import functools

import jax
import jax.numpy as jnp
from jax import lax
from jax.experimental import pallas as pl
from jax.experimental.pallas import tpu as pltpu
from jax.experimental.pallas import tpu_sc as plsc

_B, _N, _DM = 4, 4096, 1024
_NS = 2
_SD = _DM // _NS
_K = 8192
_BETA = 0.25
_T = _B * _N

_TM = 512
_TN = 2736
_KPAD = 3 * _TN
_TB = _T // _TM
_KB = 3

_NW = 32
_TOK_W = _T // _NW
_CH = 128
_NCH = _TOK_W // _CH


def _argmin_body(xt_ref, w_ref, fs_ref, wsq_ref, ids_ref, mind_ref,
                 rmin_ref, ridx_ref):
    k = pl.program_id(0)
    t = pl.program_id(1)
    mm = lax.dot_general(
        w_ref[...], xt_ref[...],
        dimension_numbers=(((1,), (0,)), ((), ())),
        preferred_element_type=jnp.float32)
    t1 = wsq_ref[:, :] + fs_ref[0]
    d = t1 - mm
    bmin = jnp.min(d, axis=0, keepdims=True)
    gidx = jax.lax.broadcasted_iota(jnp.int32, (_TN, _TM), 0) + k * _TN
    cand = jnp.where(d == bmin, gidx, jnp.int32(_KPAD))
    bidx = jnp.min(cand, axis=0, keepdims=True)

    is_first = k == 0
    is_last = k == pl.num_programs(0) - 1
    sl = pl.ds(t * _TM, _TM)
    prev_min = rmin_ref[:, sl]
    prev_idx = ridx_ref[:, sl]
    take_new = jnp.logical_or(is_first, bmin < prev_min)
    val = jnp.where(take_new, bmin, prev_min)
    idx = jnp.where(take_new, bidx, prev_idx)
    rounded = val.astype(jnp.bfloat16).astype(jnp.float32)
    rmin_ref[:, sl] = jnp.where(is_last, val, rounded)
    ridx_ref[:, sl] = idx

    @pl.when(is_last)
    def _():
        ids_ref[0] = idx
        mind_ref[0] = val


def _argmin_call(xt_s, w_p, fs3, wsq2):
    return pl.pallas_call(
        _argmin_body,
        grid=(_KB, _TB),
        in_specs=[
            pl.BlockSpec((_SD, _TM), lambda k, t: (0, t)),
            pl.BlockSpec((_TN, _SD), lambda k, t: (k, 0)),
            pl.BlockSpec((1, 1, _TM), lambda k, t: (t, 0, 0)),
            pl.BlockSpec((_TN, 1), lambda k, t: (k, 0)),
        ],
        out_specs=[
            pl.BlockSpec((1, 1, _TM), lambda k, t: (t, 0, 0)),
            pl.BlockSpec((1, 1, _TM), lambda k, t: (t, 0, 0)),
        ],
        out_shape=[
            jax.ShapeDtypeStruct((_TB, 1, _TM), jnp.int32),
            jax.ShapeDtypeStruct((_TB, 1, _TM), jnp.float32),
        ],
        compiler_params=pltpu.CompilerParams(
            dimension_semantics=("arbitrary", "arbitrary")),
        scratch_shapes=[
            pltpu.VMEM((1, _T), jnp.float32),
            pltpu.VMEM((1, _T), jnp.int32),
        ],
    )(xt_s, w_p, fs3, wsq2)


def _gather_sc(w, ids):
    mesh = plsc.VectorSubcoreMesh(core_axis_name="c", subcore_axis_name="s")

    @functools.partial(
        pl.kernel,
        mesh=mesh,
        out_type=jax.ShapeDtypeStruct((_T, _SD), jnp.float32),
        scratch_types=[
            pltpu.VMEM((_CH,), jnp.int32),
            pltpu.VMEM((_CH, _SD), jnp.float32),
            pltpu.SemaphoreType.DMA,
        ],
    )
    def gk(w_hbm, i_hbm, z_hbm, idx_v, rows_v, sem):
        wid = lax.axis_index("s") * 2 + lax.axis_index("c")
        base = wid * _TOK_W
        for c in range(_NCH):
            off = base + c * _CH
            pltpu.sync_copy(i_hbm.at[pl.ds(off, _CH)], idx_v)
            pltpu.async_copy(w_hbm.at[idx_v], rows_v, sem).wait()
            pltpu.sync_copy(rows_v, z_hbm.at[pl.ds(off, _CH)])

    return gk(w, ids)


def _st_body(h_ref, z0_ref, z1_ref, z_ref):
    h0 = h_ref[:, :_SD]
    h1 = h_ref[:, _SD:]
    z_ref[:, :_SD] = h0 + (z0_ref[...] - h0)
    z_ref[:, _SD:] = h1 + (z1_ref[...] - h1)


def _st_call(h2, z0, z1):
    tmz = 1024
    return pl.pallas_call(
        _st_body,
        grid=(_T // tmz,),
        in_specs=[
            pl.BlockSpec((tmz, _DM), lambda i: (i, 0)),
            pl.BlockSpec((tmz, _SD), lambda i: (i, 0)),
            pl.BlockSpec((tmz, _SD), lambda i: (i, 0)),
        ],
        out_specs=pl.BlockSpec((tmz, _DM), lambda i: (i, 0)),
        out_shape=jax.ShapeDtypeStruct((_T, _DM), jnp.float32),
    )(h2, z0, z1)


def kernel(h, W0, W1):
    hf = h.reshape(_T, _NS, _SD)
    xt = jnp.transpose(hf, (1, 2, 0))
    w0p = jnp.pad(2.0 * W0, ((0, _KPAD - _K), (0, 0)))
    w1p = jnp.pad(2.0 * W1, ((0, _KPAD - _K), (0, 0)))
    f0 = jnp.sum(hf[:, 0, :] ** 2, axis=1)
    f1 = jnp.sum(hf[:, 1, :] ** 2, axis=1)
    wsq0 = jnp.pad(jnp.sum(W0 ** 2, axis=1), (0, _KPAD - _K),
                   constant_values=jnp.inf).reshape(_KPAD, 1)
    wsq1 = jnp.pad(jnp.sum(W1 ** 2, axis=1), (0, _KPAD - _K),
                   constant_values=jnp.inf).reshape(_KPAD, 1)

    ids0_3, mind0 = _argmin_call(xt[0], w0p, f0.reshape(_TB, 1, _TM), wsq0)
    ids0 = ids0_3.reshape(_T)
    z0 = _gather_sc(W0, ids0)
    ids1_3, mind1 = _argmin_call(xt[1], w1p, f1.reshape(_TB, 1, _TM), wsq1)
    ids1 = ids1_3.reshape(_T)
    z1 = _gather_sc(W1, ids1)

    z = _st_call(h.reshape(_T, _DM), z0, z1).reshape(_B, _N, _DM)
    ids_packed = (ids0 + _K * ids1).reshape(_B, _N)
    vq_total = (1.0 + _BETA) * ((jnp.sum(mind0) + jnp.sum(mind1))
                                / (_T * _SD))
    return (z, ids_packed, vq_total)


# speedup vs baseline: 1.1657x; 1.1175x over previous
import functools

import jax
import jax.numpy as jnp
from jax import lax
from jax.experimental import pallas as pl
from jax.experimental.pallas import tpu as pltpu
from jax.experimental.pallas import tpu_sc as plsc

_B, _N, _DM = 4, 4096, 1024
_NS = 2
_SD = _DM // _NS
_K = 8192
_BETA = 0.25
_T = _B * _N

_TM = 1024
_TN = 2736
_KPAD = 3 * _TN
_TB = _T // _TM
_KB = 3

_NW = 32
_TOK_W = _T // _NW
_CH = 128
_NCH = _TOK_W // _CH


def _argmin_body(xt_ref, w_ref, fs_ref, wsq_ref, ids_ref, mind_ref,
                 rmin_ref, ridx_ref):
    k = pl.program_id(0)
    t = pl.program_id(1)
    mm = lax.dot_general(
        w_ref[...], xt_ref[...],
        dimension_numbers=(((1,), (0,)), ((), ())),
        preferred_element_type=jnp.float32)
    t1 = wsq_ref[:, :] + fs_ref[0]
    d = t1 - mm
    bmin = jnp.min(d, axis=0, keepdims=True)
    gidx = jax.lax.broadcasted_iota(jnp.int32, (_TN, _TM), 0)
    cand = jnp.where(d == bmin, gidx, jnp.int32(_KPAD))
    bidx = jnp.min(cand, axis=0, keepdims=True) + k * _TN

    is_first = k == 0
    is_last = k == pl.num_programs(0) - 1
    sl = pl.ds(t * _TM, _TM)
    prev_min = rmin_ref[:, sl]
    prev_idx = ridx_ref[:, sl]
    take_new = jnp.logical_or(is_first, bmin < prev_min)
    val = jnp.where(take_new, bmin, prev_min)
    idx = jnp.where(take_new, bidx, prev_idx)
    rounded = val.astype(jnp.bfloat16).astype(jnp.float32)
    rmin_ref[:, sl] = jnp.where(is_last, val, rounded)
    ridx_ref[:, sl] = idx

    @pl.when(is_last)
    def _():
        ids_ref[0] = idx
        mind_ref[0] = val


def _argmin_call(xt_s, w_p, fs3, wsq2):
    return pl.pallas_call(
        _argmin_body,
        grid=(_KB, _TB),
        in_specs=[
            pl.BlockSpec((_SD, _TM), lambda k, t: (0, t)),
            pl.BlockSpec((_TN, _SD), lambda k, t: (k, 0)),
            pl.BlockSpec((1, 1, _TM), lambda k, t: (t, 0, 0)),
            pl.BlockSpec((_TN, 1), lambda k, t: (k, 0)),
        ],
        out_specs=[
            pl.BlockSpec((1, 1, _TM), lambda k, t: (t, 0, 0)),
            pl.BlockSpec((1, 1, _TM), lambda k, t: (t, 0, 0)),
        ],
        out_shape=[
            jax.ShapeDtypeStruct((_TB, 1, _TM), jnp.int32),
            jax.ShapeDtypeStruct((_TB, 1, _TM), jnp.float32),
        ],
        compiler_params=pltpu.CompilerParams(
            dimension_semantics=("arbitrary", "arbitrary")),
        scratch_shapes=[
            pltpu.VMEM((1, _T), jnp.float32),
            pltpu.VMEM((1, _T), jnp.int32),
        ],
    )(xt_s, w_p, fs3, wsq2)


def _gather_sc(w, ids):
    mesh = plsc.VectorSubcoreMesh(core_axis_name="c", subcore_axis_name="s")

    @functools.partial(
        pl.kernel,
        mesh=mesh,
        out_type=jax.ShapeDtypeStruct((_T, _SD), jnp.float32),
        scratch_types=[
            pltpu.VMEM((_CH,), jnp.int32),
            pltpu.VMEM((_CH, _SD), jnp.float32),
            pltpu.SemaphoreType.DMA,
        ],
    )
    def gk(w_hbm, i_hbm, z_hbm, idx_v, rows_v, sem):
        wid = lax.axis_index("s") * 2 + lax.axis_index("c")
        base = wid * _TOK_W
        for c in range(_NCH):
            off = base + c * _CH
            pltpu.sync_copy(i_hbm.at[pl.ds(off, _CH)], idx_v)
            pltpu.async_copy(w_hbm.at[idx_v], rows_v, sem).wait()
            pltpu.sync_copy(rows_v, z_hbm.at[pl.ds(off, _CH)])

    return gk(w, ids)


def _st_body(h_ref, z0_ref, z1_ref, z_ref):
    h0 = h_ref[:, :_SD]
    h1 = h_ref[:, _SD:]
    z_ref[:, :_SD] = h0 + (z0_ref[...] - h0)
    z_ref[:, _SD:] = h1 + (z1_ref[...] - h1)


def _st_call(h2, z0, z1):
    tmz = 1024
    return pl.pallas_call(
        _st_body,
        grid=(_T // tmz,),
        in_specs=[
            pl.BlockSpec((tmz, _DM), lambda i: (i, 0)),
            pl.BlockSpec((tmz, _SD), lambda i: (i, 0)),
            pl.BlockSpec((tmz, _SD), lambda i: (i, 0)),
        ],
        out_specs=pl.BlockSpec((tmz, _DM), lambda i: (i, 0)),
        out_shape=jax.ShapeDtypeStruct((_T, _DM), jnp.float32),
    )(h2, z0, z1)


def kernel(h, W0, W1):
    hf = h.reshape(_T, _NS, _SD)
    xt = jnp.transpose(hf, (1, 2, 0))
    w0p = jnp.pad(2.0 * W0, ((0, _KPAD - _K), (0, 0)))
    w1p = jnp.pad(2.0 * W1, ((0, _KPAD - _K), (0, 0)))
    f0 = jnp.sum(hf[:, 0, :] ** 2, axis=1)
    f1 = jnp.sum(hf[:, 1, :] ** 2, axis=1)
    wsq0 = jnp.pad(jnp.sum(W0 ** 2, axis=1), (0, _KPAD - _K),
                   constant_values=jnp.inf).reshape(_KPAD, 1)
    wsq1 = jnp.pad(jnp.sum(W1 ** 2, axis=1), (0, _KPAD - _K),
                   constant_values=jnp.inf).reshape(_KPAD, 1)

    ids0_3, mind0 = _argmin_call(xt[0], w0p, f0.reshape(_TB, 1, _TM), wsq0)
    ids0 = ids0_3.reshape(_T)
    z0 = _gather_sc(W0, ids0)
    ids1_3, mind1 = _argmin_call(xt[1], w1p, f1.reshape(_TB, 1, _TM), wsq1)
    ids1 = ids1_3.reshape(_T)
    z1 = _gather_sc(W1, ids1)

    z = _st_call(h.reshape(_T, _DM), z0, z1).reshape(_B, _N, _DM)
    ids_packed = (ids0 + _K * ids1).reshape(_B, _N)
    vq_total = (1.0 + _BETA) * ((jnp.sum(mind0) + jnp.sum(mind1))
                                / (_T * _SD))
    return (z, ids_packed, vq_total)


# speedup vs baseline: 1.1910x; 1.0217x over previous
import functools

import jax
import jax.numpy as jnp
from jax import lax
from jax.experimental import pallas as pl
from jax.experimental.pallas import tpu as pltpu
from jax.experimental.pallas import tpu_sc as plsc

_B, _N, _DM = 4, 4096, 1024
_NS = 2
_SD = _DM // _NS
_K = 8192
_BETA = 0.25
_T = _B * _N

_TM = 2048
_TN = 2736
_KPAD = 3 * _TN
_TB = _T // _TM
_KB = 3

_NW = 32
_TOK_W = _T // _NW
_CH = 128
_NCH = _TOK_W // _CH


def _argmin_body(xt_ref, w_ref, fs_ref, wsq_ref, ids_ref, mind_ref,
                 rmin_ref, ridx_ref):
    k = pl.program_id(0)
    t = pl.program_id(1)
    mm = lax.dot_general(
        w_ref[...], xt_ref[...],
        dimension_numbers=(((1,), (0,)), ((), ())),
        preferred_element_type=jnp.float32)
    t1 = wsq_ref[:, :] + fs_ref[0]
    d = t1 - mm
    bmin = jnp.min(d, axis=0, keepdims=True)
    gidx = jax.lax.broadcasted_iota(jnp.int32, (_TN, _TM), 0)
    cand = jnp.where(d == bmin, gidx, jnp.int32(_KPAD))
    bidx = jnp.min(cand, axis=0, keepdims=True) + k * _TN

    is_first = k == 0
    is_last = k == pl.num_programs(0) - 1
    sl = pl.ds(t * _TM, _TM)
    prev_min = rmin_ref[:, sl]
    prev_idx = ridx_ref[:, sl]
    take_new = jnp.logical_or(is_first, bmin < prev_min)
    val = jnp.where(take_new, bmin, prev_min)
    idx = jnp.where(take_new, bidx, prev_idx)
    rounded = val.astype(jnp.bfloat16).astype(jnp.float32)
    rmin_ref[:, sl] = jnp.where(is_last, val, rounded)
    ridx_ref[:, sl] = idx

    @pl.when(is_last)
    def _():
        ids_ref[0] = idx
        mind_ref[0] = val


def _argmin_call(xt_s, w_p, fs3, wsq2):
    return pl.pallas_call(
        _argmin_body,
        grid=(_KB, _TB),
        in_specs=[
            pl.BlockSpec((_SD, _TM), lambda k, t: (0, t)),
            pl.BlockSpec((_TN, _SD), lambda k, t: (k, 0)),
            pl.BlockSpec((1, 1, _TM), lambda k, t: (t, 0, 0)),
            pl.BlockSpec((_TN, 1), lambda k, t: (k, 0)),
        ],
        out_specs=[
            pl.BlockSpec((1, 1, _TM), lambda k, t: (t, 0, 0)),
            pl.BlockSpec((1, 1, _TM), lambda k, t: (t, 0, 0)),
        ],
        out_shape=[
            jax.ShapeDtypeStruct((_TB, 1, _TM), jnp.int32),
            jax.ShapeDtypeStruct((_TB, 1, _TM), jnp.float32),
        ],
        compiler_params=pltpu.CompilerParams(
            dimension_semantics=("arbitrary", "arbitrary")),
        scratch_shapes=[
            pltpu.VMEM((1, _T), jnp.float32),
            pltpu.VMEM((1, _T), jnp.int32),
        ],
    )(xt_s, w_p, fs3, wsq2)


def _gather_sc(w, ids):
    mesh = plsc.VectorSubcoreMesh(core_axis_name="c", subcore_axis_name="s")

    @functools.partial(
        pl.kernel,
        mesh=mesh,
        out_type=jax.ShapeDtypeStruct((_T, _SD), jnp.float32),
        scratch_types=[
            pltpu.VMEM((_CH,), jnp.int32),
            pltpu.VMEM((_CH, _SD), jnp.float32),
            pltpu.SemaphoreType.DMA,
        ],
    )
    def gk(w_hbm, i_hbm, z_hbm, idx_v, rows_v, sem):
        wid = lax.axis_index("s") * 2 + lax.axis_index("c")
        base = wid * _TOK_W
        for c in range(_NCH):
            off = base + c * _CH
            pltpu.sync_copy(i_hbm.at[pl.ds(off, _CH)], idx_v)
            pltpu.async_copy(w_hbm.at[idx_v], rows_v, sem).wait()
            pltpu.sync_copy(rows_v, z_hbm.at[pl.ds(off, _CH)])

    return gk(w, ids)


def _st_body(h_ref, z0_ref, z1_ref, z_ref):
    h0 = h_ref[:, :_SD]
    h1 = h_ref[:, _SD:]
    z_ref[:, :_SD] = h0 + (z0_ref[...] - h0)
    z_ref[:, _SD:] = h1 + (z1_ref[...] - h1)


def _st_call(h2, z0, z1):
    tmz = 1024
    return pl.pallas_call(
        _st_body,
        grid=(_T // tmz,),
        in_specs=[
            pl.BlockSpec((tmz, _DM), lambda i: (i, 0)),
            pl.BlockSpec((tmz, _SD), lambda i: (i, 0)),
            pl.BlockSpec((tmz, _SD), lambda i: (i, 0)),
        ],
        out_specs=pl.BlockSpec((tmz, _DM), lambda i: (i, 0)),
        out_shape=jax.ShapeDtypeStruct((_T, _DM), jnp.float32),
    )(h2, z0, z1)


def kernel(h, W0, W1):
    hf = h.reshape(_T, _NS, _SD)
    xt = jnp.transpose(hf, (1, 2, 0))
    w0p = jnp.pad(2.0 * W0, ((0, _KPAD - _K), (0, 0)))
    w1p = jnp.pad(2.0 * W1, ((0, _KPAD - _K), (0, 0)))
    f0 = jnp.sum(hf[:, 0, :] ** 2, axis=1)
    f1 = jnp.sum(hf[:, 1, :] ** 2, axis=1)
    wsq0 = jnp.pad(jnp.sum(W0 ** 2, axis=1), (0, _KPAD - _K),
                   constant_values=jnp.inf).reshape(_KPAD, 1)
    wsq1 = jnp.pad(jnp.sum(W1 ** 2, axis=1), (0, _KPAD - _K),
                   constant_values=jnp.inf).reshape(_KPAD, 1)

    ids0_3, mind0 = _argmin_call(xt[0], w0p, f0.reshape(_TB, 1, _TM), wsq0)
    ids0 = ids0_3.reshape(_T)
    z0 = _gather_sc(W0, ids0)
    ids1_3, mind1 = _argmin_call(xt[1], w1p, f1.reshape(_TB, 1, _TM), wsq1)
    ids1 = ids1_3.reshape(_T)
    z1 = _gather_sc(W1, ids1)

    z = _st_call(h.reshape(_T, _DM), z0, z1).reshape(_B, _N, _DM)
    ids_packed = (ids0 + _K * ids1).reshape(_B, _N)
    vq_total = (1.0 + _BETA) * ((jnp.sum(mind0) + jnp.sum(mind1))
                                / (_T * _SD))
    return (z, ids_packed, vq_total)


# speedup vs baseline: 1.2061x; 1.0127x over previous
import functools

import jax
import jax.numpy as jnp
from jax import lax
from jax.experimental import pallas as pl
from jax.experimental.pallas import tpu as pltpu
from jax.experimental.pallas import tpu_sc as plsc

_B, _N, _DM = 4, 4096, 1024
_NS = 2
_SD = _DM // _NS
_K = 8192
_BETA = 0.25
_T = _B * _N

_TM = 4096
_TN = 2736
_KPAD = 3 * _TN
_TB = _T // _TM
_KB = 3

_NW = 32
_TOK_W = _T // _NW
_CH = 128
_NCH = _TOK_W // _CH


def _argmin_body(xt_ref, w_ref, fs_ref, wsq_ref, ids_ref, mind_ref,
                 rmin_ref, ridx_ref):
    k = pl.program_id(0)
    t = pl.program_id(1)
    mm = lax.dot_general(
        w_ref[...], xt_ref[...],
        dimension_numbers=(((1,), (0,)), ((), ())),
        preferred_element_type=jnp.float32)
    t1 = wsq_ref[:, :] + fs_ref[0]
    d = t1 - mm
    bmin = jnp.min(d, axis=0, keepdims=True)
    gidx = jax.lax.broadcasted_iota(jnp.int32, (_TN, _TM), 0)
    cand = jnp.where(d == bmin, gidx, jnp.int32(_KPAD))
    bidx = jnp.min(cand, axis=0, keepdims=True) + k * _TN

    is_first = k == 0
    is_last = k == pl.num_programs(0) - 1
    sl = pl.ds(t * _TM, _TM)
    prev_min = rmin_ref[:, sl]
    prev_idx = ridx_ref[:, sl]
    take_new = jnp.logical_or(is_first, bmin < prev_min)
    val = jnp.where(take_new, bmin, prev_min)
    idx = jnp.where(take_new, bidx, prev_idx)
    rounded = val.astype(jnp.bfloat16).astype(jnp.float32)
    rmin_ref[:, sl] = jnp.where(is_last, val, rounded)
    ridx_ref[:, sl] = idx

    @pl.when(is_last)
    def _():
        ids_ref[0] = idx
        mind_ref[0] = val


def _argmin_call(xt_s, w_p, fs3, wsq2):
    return pl.pallas_call(
        _argmin_body,
        grid=(_KB, _TB),
        in_specs=[
            pl.BlockSpec((_SD, _TM), lambda k, t: (0, t)),
            pl.BlockSpec((_TN, _SD), lambda k, t: (k, 0)),
            pl.BlockSpec((1, 1, _TM), lambda k, t: (t, 0, 0)),
            pl.BlockSpec((_TN, 1), lambda k, t: (k, 0)),
        ],
        out_specs=[
            pl.BlockSpec((1, 1, _TM), lambda k, t: (t, 0, 0)),
            pl.BlockSpec((1, 1, _TM), lambda k, t: (t, 0, 0)),
        ],
        out_shape=[
            jax.ShapeDtypeStruct((_TB, 1, _TM), jnp.int32),
            jax.ShapeDtypeStruct((_TB, 1, _TM), jnp.float32),
        ],
        compiler_params=pltpu.CompilerParams(
            dimension_semantics=("arbitrary", "arbitrary")),
        scratch_shapes=[
            pltpu.VMEM((1, _T), jnp.float32),
            pltpu.VMEM((1, _T), jnp.int32),
        ],
    )(xt_s, w_p, fs3, wsq2)


def _gather_sc(w, ids):
    mesh = plsc.VectorSubcoreMesh(core_axis_name="c", subcore_axis_name="s")

    @functools.partial(
        pl.kernel,
        mesh=mesh,
        out_type=jax.ShapeDtypeStruct((_T, _SD), jnp.float32),
        scratch_types=[
            pltpu.VMEM((_CH,), jnp.int32),
            pltpu.VMEM((_CH, _SD), jnp.float32),
            pltpu.SemaphoreType.DMA,
        ],
    )
    def gk(w_hbm, i_hbm, z_hbm, idx_v, rows_v, sem):
        wid = lax.axis_index("s") * 2 + lax.axis_index("c")
        base = wid * _TOK_W
        for c in range(_NCH):
            off = base + c * _CH
            pltpu.sync_copy(i_hbm.at[pl.ds(off, _CH)], idx_v)
            pltpu.async_copy(w_hbm.at[idx_v], rows_v, sem).wait()
            pltpu.sync_copy(rows_v, z_hbm.at[pl.ds(off, _CH)])

    return gk(w, ids)


def _st_body(h_ref, z0_ref, z1_ref, z_ref):
    h0 = h_ref[:, :_SD]
    h1 = h_ref[:, _SD:]
    z_ref[:, :_SD] = h0 + (z0_ref[...] - h0)
    z_ref[:, _SD:] = h1 + (z1_ref[...] - h1)


def _st_call(h2, z0, z1):
    tmz = 1024
    return pl.pallas_call(
        _st_body,
        grid=(_T // tmz,),
        in_specs=[
            pl.BlockSpec((tmz, _DM), lambda i: (i, 0)),
            pl.BlockSpec((tmz, _SD), lambda i: (i, 0)),
            pl.BlockSpec((tmz, _SD), lambda i: (i, 0)),
        ],
        out_specs=pl.BlockSpec((tmz, _DM), lambda i: (i, 0)),
        out_shape=jax.ShapeDtypeStruct((_T, _DM), jnp.float32),
    )(h2, z0, z1)


def kernel(h, W0, W1):
    hf = h.reshape(_T, _NS, _SD)
    xt = jnp.transpose(hf, (1, 2, 0))
    w0p = jnp.pad(2.0 * W0, ((0, _KPAD - _K), (0, 0)))
    w1p = jnp.pad(2.0 * W1, ((0, _KPAD - _K), (0, 0)))
    f0 = jnp.sum(hf[:, 0, :] ** 2, axis=1)
    f1 = jnp.sum(hf[:, 1, :] ** 2, axis=1)
    wsq0 = jnp.pad(jnp.sum(W0 ** 2, axis=1), (0, _KPAD - _K),
                   constant_values=jnp.inf).reshape(_KPAD, 1)
    wsq1 = jnp.pad(jnp.sum(W1 ** 2, axis=1), (0, _KPAD - _K),
                   constant_values=jnp.inf).reshape(_KPAD, 1)

    ids0_3, mind0 = _argmin_call(xt[0], w0p, f0.reshape(_TB, 1, _TM), wsq0)
    ids0 = ids0_3.reshape(_T)
    z0 = _gather_sc(W0, ids0)
    ids1_3, mind1 = _argmin_call(xt[1], w1p, f1.reshape(_TB, 1, _TM), wsq1)
    ids1 = ids1_3.reshape(_T)
    z1 = _gather_sc(W1, ids1)

    z = _st_call(h.reshape(_T, _DM), z0, z1).reshape(_B, _N, _DM)
    ids_packed = (ids0 + _K * ids1).reshape(_B, _N)
    vq_total = (1.0 + _BETA) * ((jnp.sum(mind0) + jnp.sum(mind1))
                                / (_T * _SD))
    return (z, ids_packed, vq_total)
